# baseline (speedup 1.0000x reference)
import functools

import jax
import jax.numpy as jnp
from jax import lax
from jax.experimental import pallas as pl
from jax.experimental.pallas import tpu as pltpu
from jax.experimental.pallas import tpu_sc as plsc

BATCH = 16384
EMBED_DIM = 32
TOTAL = 2 * BATCH
PACK = 128 // EMBED_DIM
TAB_ROWS = 1000000 // PACK

_info = plsc.get_sparse_core_info()
NC, NS, L = _info.num_cores, _info.num_subcores, _info.num_lanes
NW = NC * NS
B_PER_W = TOTAL // NW
CHUNK = 128
NCHUNK = B_PER_W // CHUNK
GROUPS = CHUNK // L

_mesh = plsc.VectorSubcoreMesh(core_axis_name="c", subcore_axis_name="s")


@functools.partial(
    pl.kernel,
    mesh=_mesh,
    out_type=jax.ShapeDtypeStruct((TOTAL,), jnp.float32),
    compiler_params=pltpu.CompilerParams(
        needs_layout_passes=False, use_tc_tiling_on_sc=True
    ),
    scratch_types=[
        pltpu.VMEM((NCHUNK, CHUNK), jnp.int32),
        pltpu.VMEM((NCHUNK, CHUNK), jnp.int32),
        pltpu.VMEM((NCHUNK, CHUNK), jnp.int32),
        pltpu.VMEM((NCHUNK, CHUNK), jnp.int32),
        pltpu.VMEM((NCHUNK, CHUNK), jnp.int32),
        pltpu.VMEM((NCHUNK, CHUNK), jnp.int32),
        pltpu.VMEM((CHUNK, 128), jnp.float32),
        pltpu.VMEM((CHUNK, 128), jnp.float32),
        pltpu.VMEM((CHUNK, 128), jnp.float32),
        pltpu.VMEM((CHUNK, 128), jnp.float32),
        pltpu.VMEM((CHUNK, 128), jnp.float32),
        pltpu.VMEM((CHUNK, 128), jnp.float32),
        pltpu.VMEM((B_PER_W,), jnp.float32),
        pltpu.SemaphoreType.DMA,
    ],
)
def _complex_score_kernel(
    s_hbm, r_hbm, o_hbm, so_hbm, ro_hbm, oo_hbm,
    ent_real, ent_imag, rel_real, rel_imag,
    out_hbm,
    s_v, r_v, o_v, so_v, ro_v, oo_v,
    sr_v, si_v, or_v, oi_v, rr_v, ri_v,
    scores_v, sem,
):
    wid = lax.axis_index("s") * NC + lax.axis_index("c")

    pltpu.sync_copy(s_hbm.at[wid], s_v)
    pltpu.sync_copy(r_hbm.at[wid], r_v)
    pltpu.sync_copy(o_hbm.at[wid], o_v)
    pltpu.sync_copy(so_hbm.at[wid], so_v)
    pltpu.sync_copy(ro_hbm.at[wid], ro_v)
    pltpu.sync_copy(oo_hbm.at[wid], oo_v)

    lane = lax.iota(jnp.int32, L)

    def chunk_body(g, carry):
        cps = [
            pltpu.async_copy(ent_real.at[s_v.at[g]], sr_v, sem),
            pltpu.async_copy(ent_imag.at[s_v.at[g]], si_v, sem),
            pltpu.async_copy(ent_real.at[o_v.at[g]], or_v, sem),
            pltpu.async_copy(ent_imag.at[o_v.at[g]], oi_v, sem),
            pltpu.async_copy(rel_real.at[r_v.at[g]], rr_v, sem),
            pltpu.async_copy(rel_imag.at[r_v.at[g]], ri_v, sem),
        ]
        for cp in cps:
            cp.wait()

        def group_body(g2, carry2):
            rows = g2 * L + lane
            offs = so_v[g, pl.ds(g2 * L, L)]
            offr = ro_v[g, pl.ds(g2 * L, L)]
            offo = oo_v[g, pl.ds(g2 * L, L)]
            acc = jnp.zeros((L,), jnp.float32)
            for d in range(EMBED_DIM):
                cs = offs + d
                cr = offr + d
                co = offo + d
                sr = plsc.load_gather(sr_v, [rows, cs])
                si = plsc.load_gather(si_v, [rows, cs])
                orr = plsc.load_gather(or_v, [rows, co])
                oii = plsc.load_gather(oi_v, [rows, co])
                rr = plsc.load_gather(rr_v, [rows, cr])
                ri = plsc.load_gather(ri_v, [rows, cr])
                acc = acc + (sr * (orr + oii) + si * (oii - orr) + (rr + ri))
            scores_v[pl.ds(g * CHUNK + g2 * L, L)] = acc
            return carry2

        lax.fori_loop(0, GROUPS, group_body, 0)
        return carry

    lax.fori_loop(0, NCHUNK, chunk_body, 0)

    pltpu.sync_copy(scores_v, out_hbm.at[pl.ds(wid * B_PER_W, B_PER_W)])


def kernel(positive, negative, ent_real, ent_imag, rel_real, rel_imag):
    trip = jnp.concatenate([positive, negative], axis=0)
    rows = (trip // PACK).astype(jnp.int32)
    offs = ((trip % PACK) * EMBED_DIM).astype(jnp.int32)
    s_idx = rows[:, 0].reshape(NW, NCHUNK, CHUNK)
    r_idx = rows[:, 1].reshape(NW, NCHUNK, CHUNK)
    o_idx = rows[:, 2].reshape(NW, NCHUNK, CHUNK)
    s_off = offs[:, 0].reshape(NW, NCHUNK, CHUNK)
    r_off = offs[:, 1].reshape(NW, NCHUNK, CHUNK)
    o_off = offs[:, 2].reshape(NW, NCHUNK, CHUNK)
    er = ent_real.reshape(TAB_ROWS, 128)
    ei = ent_imag.reshape(TAB_ROWS, 128)
    rr = rel_real.reshape(TAB_ROWS, 128)
    ri = rel_imag.reshape(TAB_ROWS, 128)
    out = _complex_score_kernel(
        s_idx, r_idx, o_idx, s_off, r_off, o_off, er, ei, rr, ri
    )
    return out[:BATCH], out[BATCH:]
